# baseline (speedup 1.0000x reference)
import functools

import jax
import jax.numpy as jnp
import numpy as np
from jax import lax
from jax.experimental import pallas as pl
from jax.experimental.pallas import tpu as pltpu
from jax.experimental.pallas import tpu_sc as plsc

N = 10000
E = 320000
BAG = 32
IN_CH = 128
HID = 128
NUM_GRAPHS = 16

NC = 2
NS = 16
CH = 80

EMB_CHUNKS = (N * BAG) // (NC * NS * CH)

PCH = 125
PROP_CHUNKS = E // (NS * PCH)

KB = 25
KP = 20
NB = PROP_CHUNKS // KP
NBE = EMB_CHUNKS // KB

R = 1000
GRID = N // R

_MESH = plsc.VectorSubcoreMesh(core_axis_name="c", subcore_axis_name="s")


def _zero_zbuf(zbuf):
    nrow = zbuf.shape[0]

    def body(i, _):
        for k in range(zbuf.shape[1] // 16):
            zbuf[i, pl.ds(k * 16, 16)] = jnp.zeros((16,), jnp.float32)
        return 0

    lax.fori_loop(0, nrow, body, 0)


@functools.partial(
    pl.kernel,
    mesh=_MESH,
    out_type=[
        jax.ShapeDtypeStruct((N, IN_CH), jnp.float32),
        jax.ShapeDtypeStruct((NC * N,), jnp.float32),
    ],
    scratch_types=[
        pltpu.VMEM((KB * CH,), jnp.int32),
        pltpu.VMEM((KB, CH), jnp.int32),
        pltpu.VMEM((CH,), jnp.int32),
        pltpu.VMEM((CH,), jnp.float32),
        pltpu.VMEM((CH, IN_CH), jnp.float32),
        pltpu.VMEM((CH, IN_CH), jnp.float32),
        pltpu.VMEM((2000,), jnp.float32),
        pltpu.VMEM_SHARED((N // NC, IN_CH), jnp.float32),
        pltpu.VMEM_SHARED((N,), jnp.float32),
        pltpu.SemaphoreType.DMA,
        pltpu.SemaphoreType.DMA,
    ],
)
def _sc_embdeg(xflat_hbm, dst3d_hbm, cb_hbm, emb_hbm, degp_hbm,
               bagidx_v, dstidx_v, sid_v, ones_v, rowsa_v, rowsb_v, zbuf1_v,
               emb_sh, deg_sh, sema, semb):
    c = lax.axis_index("c")
    s = lax.axis_index("s")
    rows_per_tile = N * BAG // (NC * NS)

    _zero_zbuf(rowsa_v)

    def z1(i, _):
        zbuf1_v[pl.ds(i * 16, 16)] = jnp.zeros((16,), jnp.float32)
        return 0

    lax.fori_loop(0, 125, z1, 0)

    def o1(i, _):
        ones_v[pl.ds(i * 16, 16)] = jnp.full((16,), 1.0, jnp.float32)
        return 0

    lax.fori_loop(0, CH // 16, o1, 0)

    zsrc = rowsa_v.at[pl.ds(0, 40)]
    for b in range(8):
        blk = s * 8 + b

        @pl.when(blk < 125)
        def _():
            pltpu.sync_copy(zsrc, emb_sh.at[pl.ds(blk * 40, 40)])

    @pl.when(s < 5)
    def _():
        pltpu.sync_copy(zbuf1_v, deg_sh.at[pl.ds(s * 2000, 2000)])

    plsc.subcore_barrier()

    row_off = s * rows_per_tile
    flat_off = (c * NS + s) * rows_per_tile

    def gather(j, rows, sem):
        pltpu.async_copy(cb_hbm.at[bagidx_v.at[pl.ds(j * CH, CH)]], rows, sem)

    def drain_scatter(b, j, rows, sem):
        pltpu.make_async_copy(cb_hbm.at[bagidx_v.at[pl.ds(j * CH, CH)]], rows,
                              sem).wait()
        base = row_off + (b * KB + j) * CH
        for k in range(CH // 16):
            ids = (base + k * 16 + lax.iota(jnp.int32, 16)) >> 5
            sid_v[pl.ds(k * 16, 16)] = ids
        pltpu.sync_copy(rows, emb_sh.at[sid_v], add=True)

    def eblock(b, _):
        pltpu.sync_copy(
            xflat_hbm.at[pl.ds(flat_off + b * (KB * CH), KB * CH)], bagidx_v)

        gather(0, rowsa_v, sema)

        def body(t, _):
            gather(2 * t + 1, rowsb_v, semb)
            drain_scatter(b, 2 * t, rowsa_v, sema)
            gather(2 * t + 2, rowsa_v, sema)
            drain_scatter(b, 2 * t + 1, rowsb_v, semb)
            return 0

        lax.fori_loop(0, KB // 2, body, 0)
        drain_scatter(b, KB - 1, rowsa_v, sema)
        return 0

    lax.fori_loop(0, NBE, eblock, 0)

    def dblock(b, _):
        pltpu.sync_copy(dst3d_hbm.at[(c * NS + s) * NBE + b], dstidx_v)

        def dbody(j, _):
            pltpu.sync_copy(ones_v, deg_sh.at[dstidx_v.at[j]], add=True)
            return 0

        lax.fori_loop(0, KB, dbody, 0)
        return 0

    lax.fori_loop(0, NBE, dblock, 0)

    plsc.subcore_barrier()

    for b in range(2):
        blk = s * 2 + b

        @pl.when(blk < 25)
        def _():
            pltpu.sync_copy(emb_sh.at[pl.ds(blk * 200, 200)],
                            emb_hbm.at[pl.ds(c * (N // NC) + blk * 200, 200)])

    @pl.when(s < 5)
    def _():
        pltpu.sync_copy(deg_sh.at[pl.ds(s * 2000, 2000)], zbuf1_v)
        pltpu.sync_copy(zbuf1_v, degp_hbm.at[pl.ds(c * N + s * 2000, 2000)])


@functools.partial(
    pl.kernel,
    mesh=_MESH,
    out_type=jax.ShapeDtypeStruct((NC, N, HID), jnp.float32),
    scratch_types=[
        pltpu.VMEM((KP, PCH), jnp.int32),
        pltpu.VMEM((KP, PCH), jnp.int32),
        pltpu.VMEM((PCH, HID), jnp.float32),
        pltpu.VMEM((PCH, HID), jnp.float32),
        pltpu.VMEM_SHARED((N, HID), jnp.float32),
        pltpu.SemaphoreType.DMA,
        pltpu.SemaphoreType.DMA,
    ],
)
def _sc_prop(src4d_hbm, dst4d_hbm, h_hbm, acc_hbm,
             sidx_v, didx_v, rowsa_v, rowsb_v, acc_sh, sema, semb):
    c = lax.axis_index("c")
    s = lax.axis_index("s")

    _zero_zbuf(rowsa_v)
    zsrc = rowsa_v.at[pl.ds(0, 80)]
    for b in range(8):
        blk = s * 8 + b

        @pl.when(blk < 125)
        def _():
            pltpu.sync_copy(zsrc, acc_sh.at[pl.ds(blk * 80, 80)])

    plsc.subcore_barrier()

    hc = h_hbm.at[c]

    def gather(j, rows, sem):
        pltpu.async_copy(hc.at[sidx_v.at[j]], rows, sem)

    def drain_scatter(j, rows, sem):
        pltpu.make_async_copy(hc.at[sidx_v.at[j]], rows, sem).wait()
        pltpu.sync_copy(rows, acc_sh.at[didx_v.at[j]], add=True)

    def block(b, _):
        pltpu.sync_copy(src4d_hbm.at[s].at[b], sidx_v)
        pltpu.sync_copy(dst4d_hbm.at[s].at[b], didx_v)

        gather(0, rowsa_v, sema)

        def body(t, _):
            gather(2 * t + 1, rowsb_v, semb)
            drain_scatter(2 * t, rowsa_v, sema)
            gather(2 * t + 2, rowsa_v, sema)
            drain_scatter(2 * t + 1, rowsb_v, semb)
            return 0

        lax.fori_loop(0, KP // 2 - 1, body, 0)
        gather(KP - 1, rowsb_v, semb)
        drain_scatter(KP - 2, rowsa_v, sema)
        drain_scatter(KP - 1, rowsb_v, semb)
        return 0

    lax.fori_loop(0, NB, block, 0)

    plsc.subcore_barrier()

    for b in range(4):
        blk = s * 4 + b

        @pl.when(blk < 50)
        def _():
            pltpu.sync_copy(acc_sh.at[pl.ds(blk * 200, 200)],
                            acc_hbm.at[c, pl.ds(blk * 200, 200)])


def _tc1_body(e_ref, dg_ref, w1_ref, w2_ref, h_ref, dv_ref):
    dsum = dg_ref[:, 0:1] + dg_ref[:, 1:2] + 1.0
    dinv = lax.rsqrt(jnp.maximum(dsum, 1.0))
    dv_ref[...] = dinv
    sc = dinv * (1.0 / BAG)
    e = e_ref[...]
    h_ref[0] = jnp.dot(e, w1_ref[...], preferred_element_type=jnp.float32) * sc
    h_ref[1] = jnp.dot(e, w2_ref[...], preferred_element_type=jnp.float32) * sc


def _tc1(emb, degt, w11, w21):
    return pl.pallas_call(
        _tc1_body,
        grid=(GRID,),
        in_specs=[
            pl.BlockSpec((R, IN_CH), lambda i: (i, 0)),
            pl.BlockSpec((R, 2), lambda i: (i, 0)),
            pl.BlockSpec((IN_CH, HID), lambda i: (0, 0)),
            pl.BlockSpec((IN_CH, HID), lambda i: (0, 0)),
        ],
        out_specs=[
            pl.BlockSpec((NC, R, HID), lambda i: (0, i, 0)),
            pl.BlockSpec((R, 1), lambda i: (i, 0)),
        ],
        out_shape=[
            jax.ShapeDtypeStruct((NC, N, HID), jnp.float32),
            jax.ShapeDtypeStruct((N, 1), jnp.float32),
        ],
    )(emb, degt, w11, w21)


def _prelu(t, a):
    return jnp.where(t > 0, t, a * t)


def _tc2_body(acc_ref, h_ref, dv_ref, b_ref,
              b1_ref, a1_ref, w1_ref, b2_ref, a2_ref, w2_ref,
              hn_ref, p1_ref, p2_ref):
    i = pl.program_id(0)
    dinv = dv_ref[...]
    bb = b_ref[...].reshape(1, R)
    oh = (lax.broadcasted_iota(jnp.int32, (NUM_GRAPHS, R), 0)
          == bb).astype(jnp.float32)

    z1 = _prelu((acc_ref[0] + h_ref[0]) * dinv + b1_ref[...], a1_ref[...])
    hn_ref[0] = jnp.dot(z1, w1_ref[...], preferred_element_type=jnp.float32) * dinv

    z2 = _prelu((acc_ref[1] + h_ref[1]) * dinv + b2_ref[...], a2_ref[...])
    hn_ref[1] = jnp.dot(z2, w2_ref[...], preferred_element_type=jnp.float32) * dinv

    @pl.when(i == 0)
    def _():
        p1_ref[...] = jnp.zeros_like(p1_ref)
        p2_ref[...] = jnp.zeros_like(p2_ref)

    p1_ref[...] += jnp.dot(oh, z1, preferred_element_type=jnp.float32)
    p2_ref[...] += jnp.dot(oh, z2, preferred_element_type=jnp.float32)


def _tc2(acc, h, dinv, batch2d, b1, a1, w1, b2, a2, w2):
    full = lambda shape: pl.BlockSpec(shape, lambda i: tuple(0 for _ in shape))
    return pl.pallas_call(
        _tc2_body,
        grid=(GRID,),
        in_specs=[
            pl.BlockSpec((NC, R, HID), lambda i: (0, i, 0)),
            pl.BlockSpec((NC, R, HID), lambda i: (0, i, 0)),
            pl.BlockSpec((R, 1), lambda i: (i, 0)),
            pl.BlockSpec((1, 1, R), lambda i: (i, 0, 0)),
            full((1, HID)), full((1, HID)), full((HID, HID)),
            full((1, HID)), full((1, HID)), full((HID, HID)),
        ],
        out_specs=[
            pl.BlockSpec((NC, R, HID), lambda i: (0, i, 0)),
            full((NUM_GRAPHS, HID)),
            full((NUM_GRAPHS, HID)),
        ],
        out_shape=[
            jax.ShapeDtypeStruct((NC, N, HID), jnp.float32),
            jax.ShapeDtypeStruct((NUM_GRAPHS, HID), jnp.float32),
            jax.ShapeDtypeStruct((NUM_GRAPHS, HID), jnp.float32),
        ],
    )(acc, h, dinv, batch2d, b1, a1, w1, b2, a2, w2)


def _mlp_apply(z, w1, b1, aa1, w2, b2, aa2, w3, b3, aa3, ws, bs):
    h = _prelu(jnp.dot(z, w1, preferred_element_type=jnp.float32) + b1, aa1)
    h = _prelu(jnp.dot(h, w2, preferred_element_type=jnp.float32) + b2, aa2)
    h = _prelu(jnp.dot(h, w3, preferred_element_type=jnp.float32) + b3, aa3)
    return h + jnp.dot(z, ws, preferred_element_type=jnp.float32) + bs


def _tc3_body(acc_ref, h_ref, dv_ref, b_ref, b1_ref, a1_ref, b2_ref, a2_ref,
              p11_ref, p21_ref,
              mw1, mb1, ma1, mw2, mb2, ma2, mw3, mb3, ma3, mws, mbs,
              gw1, gb1, ga1, gw2, gb2, ga2, gw3, gb3, ga3, gws, gbs,
              lv1_ref, lv2_ref, gv1_ref, gv2_ref, p1_ref, p2_ref):
    i = pl.program_id(0)
    dinv = dv_ref[...]
    bb = b_ref[...].reshape(1, R)
    oh = (lax.broadcasted_iota(jnp.int32, (NUM_GRAPHS, R), 0)
          == bb).astype(jnp.float32)

    z1 = _prelu((acc_ref[0] + h_ref[0]) * dinv + b1_ref[...], a1_ref[...])
    z2 = _prelu((acc_ref[1] + h_ref[1]) * dinv + b2_ref[...], a2_ref[...])

    @pl.when(i == 0)
    def _():
        p1_ref[...] = jnp.zeros_like(p1_ref)
        p2_ref[...] = jnp.zeros_like(p2_ref)

    p1_ref[...] += jnp.dot(oh, z1, preferred_element_type=jnp.float32)
    p2_ref[...] += jnp.dot(oh, z2, preferred_element_type=jnp.float32)

    args = (mw1[...], mb1[...], ma1[...], mw2[...], mb2[...], ma2[...],
            mw3[...], mb3[...], ma3[...], mws[...], mbs[...])
    lv1_ref[...] = _mlp_apply(z1, *args)
    lv2_ref[...] = _mlp_apply(z2, *args)

    @pl.when(i == GRID - 1)
    def _():
        gargs = (gw1[...], gb1[...], ga1[...], gw2[...], gb2[...], ga2[...],
                 gw3[...], gb3[...], ga3[...], gws[...], gbs[...])
        g1 = jnp.concatenate([p11_ref[...], p1_ref[...]], axis=1)
        g2 = jnp.concatenate([p21_ref[...], p2_ref[...]], axis=1)
        gv1_ref[...] = _mlp_apply(g1, *gargs)
        gv2_ref[...] = _mlp_apply(g2, *gargs)


def _tc3(acc, h, dinv, batch2d, b1, a1, b2, a2, p11, p21, m, gm):
    full = lambda shape: pl.BlockSpec(shape, lambda i: tuple(0 for _ in shape))
    mlp_specs = lambda k: (
        full((k * HID, HID)), full((1, HID)), full((1, 1)),
        full((HID, HID)), full((1, HID)), full((1, 1)),
        full((HID, HID)), full((1, HID)), full((1, 1)),
        full((k * HID, HID)), full((1, HID)))
    return pl.pallas_call(
        _tc3_body,
        grid=(GRID,),
        in_specs=[
            pl.BlockSpec((NC, R, HID), lambda i: (0, i, 0)),
            pl.BlockSpec((NC, R, HID), lambda i: (0, i, 0)),
            pl.BlockSpec((R, 1), lambda i: (i, 0)),
            pl.BlockSpec((1, 1, R), lambda i: (i, 0, 0)),
            full((1, HID)), full((1, HID)), full((1, HID)), full((1, HID)),
            full((NUM_GRAPHS, HID)), full((NUM_GRAPHS, HID)),
            *mlp_specs(1), *mlp_specs(2),
        ],
        out_specs=[
            pl.BlockSpec((R, HID), lambda i: (i, 0)),
            pl.BlockSpec((R, HID), lambda i: (i, 0)),
            full((NUM_GRAPHS, HID)),
            full((NUM_GRAPHS, HID)),
            full((NUM_GRAPHS, HID)),
            full((NUM_GRAPHS, HID)),
        ],
        out_shape=[
            jax.ShapeDtypeStruct((N, HID), jnp.float32),
            jax.ShapeDtypeStruct((N, HID), jnp.float32),
            jax.ShapeDtypeStruct((NUM_GRAPHS, HID), jnp.float32),
            jax.ShapeDtypeStruct((NUM_GRAPHS, HID), jnp.float32),
            jax.ShapeDtypeStruct((NUM_GRAPHS, HID), jnp.float32),
            jax.ShapeDtypeStruct((NUM_GRAPHS, HID), jnp.float32),
        ],
    )(acc, h, dinv, batch2d, b1, a1, b2, a2, p11, p21, *m, *gm)


def _mlp_args(p, in_ft):
    r = lambda v: v.reshape(1, -1)
    sc = lambda v: v.reshape(1, 1)
    return (p['W1'], r(p['b1']), sc(p['a1']),
            p['W2'], r(p['b2']), sc(p['a2']),
            p['W3'], r(p['b3']), sc(p['a3']),
            p['Ws'], r(p['bs']))


def kernel(x, edge_index, batch, params):
    p = params
    cb = p['codebook']

    xflat = x.reshape(N * BAG)
    src4d = edge_index[0].reshape(NS, NB, KP, PCH)
    dstflat = edge_index[1]
    dst4d = dstflat.reshape(NS, NB, KP, PCH)
    dst3d = dstflat.reshape(NC * NS * NBE, KB, CH)
    batchr = batch.reshape(GRID, 1, R)

    emb, degp = _sc_embdeg(xflat, dst3d, cb)
    degt = degp.reshape(NC, N).T

    g1, g2 = p['gnn1'], p['gnn2']
    (w11, b11), (w12, b12) = g1['layers']
    (w21, b21), (w22, b22) = g2['layers']
    a1 = g1['a'].reshape(1, HID)
    a2 = g2['a'].reshape(1, HID)
    rb = lambda v: v.reshape(1, HID)

    h, dinv = _tc1(emb, degt, w11, w21)
    acc = _sc_prop(src4d, dst4d, h)
    hn, p11, p21 = _tc2(acc, h, dinv, batchr,
                        rb(b11), a1, w12, rb(b21), a2, w22)
    acc2 = _sc_prop(src4d, dst4d, hn)
    lv1, lv2, gv1, gv2, _, _ = _tc3(acc2, hn, dinv, batchr,
                                    rb(b12), a1, rb(b22), a2, p11, p21,
                                    _mlp_args(p['mlp1'], HID),
                                    _mlp_args(p['mlp2'], 2 * HID))
    return (lv1, gv1, lv2, gv2)


# speedup vs baseline: 1.0129x; 1.0129x over previous
import functools

import jax
import jax.numpy as jnp
import numpy as np
from jax import lax
from jax.experimental import pallas as pl
from jax.experimental.pallas import tpu as pltpu
from jax.experimental.pallas import tpu_sc as plsc

N = 10000
E = 320000
BAG = 32
IN_CH = 128
HID = 128
NUM_GRAPHS = 16

NC = 2
NS = 16
CH = 80

EMB_CHUNKS = (N * BAG) // (NC * NS * CH)

PCH = 125
PROP_CHUNKS = E // (NS * PCH)

KB = 25
KP = 20
NB = PROP_CHUNKS // KP
NBE = EMB_CHUNKS // KB

R = 1000
GRID = N // R

_MESH = plsc.VectorSubcoreMesh(core_axis_name="c", subcore_axis_name="s")


def _zero_zbuf(zbuf):
    nrow = zbuf.shape[0]

    def body(i, _):
        for k in range(zbuf.shape[1] // 16):
            zbuf[i, pl.ds(k * 16, 16)] = jnp.zeros((16,), jnp.float32)
        return 0

    lax.fori_loop(0, nrow, body, 0)


@functools.partial(
    pl.kernel,
    mesh=_MESH,
    out_type=[
        jax.ShapeDtypeStruct((N, IN_CH), jnp.float32),
        jax.ShapeDtypeStruct((NC * N,), jnp.float32),
    ],
    scratch_types=[
        pltpu.VMEM((KB * CH,), jnp.int32),
        pltpu.VMEM((KB, CH), jnp.int32),
        pltpu.VMEM((CH,), jnp.int32),
        pltpu.VMEM((CH,), jnp.float32),
        pltpu.VMEM((CH, IN_CH), jnp.float32),
        pltpu.VMEM((CH, IN_CH), jnp.float32),
        pltpu.VMEM((2000,), jnp.float32),
        pltpu.VMEM_SHARED((N // NC, IN_CH), jnp.float32),
        pltpu.VMEM_SHARED((N,), jnp.float32),
        pltpu.SemaphoreType.DMA,
        pltpu.SemaphoreType.DMA,
    ],
)
def _sc_embdeg(xflat_hbm, dst3d_hbm, cb_hbm, emb_hbm, degp_hbm,
               bagidx_v, dstidx_v, sid_v, ones_v, rowsa_v, rowsb_v, zbuf1_v,
               emb_sh, deg_sh, sema, semb):
    c = lax.axis_index("c")
    s = lax.axis_index("s")
    rows_per_tile = N * BAG // (NC * NS)

    _zero_zbuf(rowsa_v)

    def z1(i, _):
        zbuf1_v[pl.ds(i * 16, 16)] = jnp.zeros((16,), jnp.float32)
        return 0

    lax.fori_loop(0, 125, z1, 0)

    def o1(i, _):
        ones_v[pl.ds(i * 16, 16)] = jnp.full((16,), 1.0, jnp.float32)
        return 0

    lax.fori_loop(0, CH // 16, o1, 0)

    zsrc = rowsa_v.at[pl.ds(0, 40)]
    for b in range(8):
        blk = s * 8 + b

        @pl.when(blk < 125)
        def _():
            pltpu.sync_copy(zsrc, emb_sh.at[pl.ds(blk * 40, 40)])

    @pl.when(s < 5)
    def _():
        pltpu.sync_copy(zbuf1_v, deg_sh.at[pl.ds(s * 2000, 2000)])

    plsc.subcore_barrier()

    row_off = s * rows_per_tile
    flat_off = (c * NS + s) * rows_per_tile

    def gather(j, rows, sem):
        pltpu.async_copy(cb_hbm.at[bagidx_v.at[pl.ds(j * CH, CH)]], rows, sem)

    def drain_scatter(b, j, rows, sem):
        pltpu.make_async_copy(cb_hbm.at[bagidx_v.at[pl.ds(j * CH, CH)]], rows,
                              sem).wait()
        base = row_off + (b * KB + j) * CH
        for k in range(CH // 16):
            ids = (base + k * 16 + lax.iota(jnp.int32, 16)) >> 5
            sid_v[pl.ds(k * 16, 16)] = ids
        pltpu.sync_copy(rows, emb_sh.at[sid_v], add=True)

    def eblock(b, _):
        pltpu.sync_copy(
            xflat_hbm.at[pl.ds(flat_off + b * (KB * CH), KB * CH)], bagidx_v)

        gather(0, rowsa_v, sema)

        def body(t, _):
            gather(2 * t + 1, rowsb_v, semb)
            drain_scatter(b, 2 * t, rowsa_v, sema)
            gather(2 * t + 2, rowsa_v, sema)
            drain_scatter(b, 2 * t + 1, rowsb_v, semb)
            return 0

        lax.fori_loop(0, KB // 2, body, 0)
        drain_scatter(b, KB - 1, rowsa_v, sema)
        return 0

    lax.fori_loop(0, NBE, eblock, 0)

    def dblock(b, _):
        pltpu.sync_copy(dst3d_hbm.at[(c * NS + s) * NBE + b], dstidx_v)

        def dissue(j, _):
            pltpu.async_copy(ones_v, deg_sh.at[dstidx_v.at[j]], sema, add=True)
            return 0

        lax.fori_loop(0, KB, dissue, 0)

        def ddrain(j, _):
            pltpu.make_async_copy(ones_v, deg_sh.at[dstidx_v.at[j]],
                                  sema).wait()
            return 0

        lax.fori_loop(0, KB, ddrain, 0)
        return 0

    lax.fori_loop(0, NBE, dblock, 0)

    plsc.subcore_barrier()

    for b in range(2):
        blk = s * 2 + b

        @pl.when(blk < 25)
        def _():
            pltpu.sync_copy(emb_sh.at[pl.ds(blk * 200, 200)],
                            emb_hbm.at[pl.ds(c * (N // NC) + blk * 200, 200)])

    @pl.when(s < 5)
    def _():
        pltpu.sync_copy(deg_sh.at[pl.ds(s * 2000, 2000)], zbuf1_v)
        pltpu.sync_copy(zbuf1_v, degp_hbm.at[pl.ds(c * N + s * 2000, 2000)])


@functools.partial(
    pl.kernel,
    mesh=_MESH,
    out_type=jax.ShapeDtypeStruct((NC, N, HID), jnp.float32),
    scratch_types=[
        pltpu.VMEM((KP, PCH), jnp.int32),
        pltpu.VMEM((KP, PCH), jnp.int32),
        pltpu.VMEM((PCH, HID), jnp.float32),
        pltpu.VMEM((PCH, HID), jnp.float32),
        pltpu.VMEM_SHARED((N, HID), jnp.float32),
        pltpu.SemaphoreType.DMA,
        pltpu.SemaphoreType.DMA,
    ],
)
def _sc_prop(src4d_hbm, dst4d_hbm, h_hbm, acc_hbm,
             sidx_v, didx_v, rowsa_v, rowsb_v, acc_sh, sema, semb):
    c = lax.axis_index("c")
    s = lax.axis_index("s")

    _zero_zbuf(rowsa_v)
    zsrc = rowsa_v.at[pl.ds(0, 80)]
    for b in range(8):
        blk = s * 8 + b

        @pl.when(blk < 125)
        def _():
            pltpu.sync_copy(zsrc, acc_sh.at[pl.ds(blk * 80, 80)])

    plsc.subcore_barrier()

    hc = h_hbm.at[c]

    def gather(j, rows, sem):
        pltpu.async_copy(hc.at[sidx_v.at[j]], rows, sem)

    def drain_scatter(j, rows, sem):
        pltpu.make_async_copy(hc.at[sidx_v.at[j]], rows, sem).wait()
        pltpu.sync_copy(rows, acc_sh.at[didx_v.at[j]], add=True)

    def block(b, _):
        pltpu.sync_copy(src4d_hbm.at[s].at[b], sidx_v)
        pltpu.sync_copy(dst4d_hbm.at[s].at[b], didx_v)

        gather(0, rowsa_v, sema)

        def body(t, _):
            gather(2 * t + 1, rowsb_v, semb)
            drain_scatter(2 * t, rowsa_v, sema)
            gather(2 * t + 2, rowsa_v, sema)
            drain_scatter(2 * t + 1, rowsb_v, semb)
            return 0

        lax.fori_loop(0, KP // 2 - 1, body, 0)
        gather(KP - 1, rowsb_v, semb)
        drain_scatter(KP - 2, rowsa_v, sema)
        drain_scatter(KP - 1, rowsb_v, semb)
        return 0

    lax.fori_loop(0, NB, block, 0)

    plsc.subcore_barrier()

    for b in range(4):
        blk = s * 4 + b

        @pl.when(blk < 50)
        def _():
            pltpu.sync_copy(acc_sh.at[pl.ds(blk * 200, 200)],
                            acc_hbm.at[c, pl.ds(blk * 200, 200)])


def _tc1_body(e_ref, dg_ref, w1_ref, w2_ref, h_ref, dv_ref):
    dsum = dg_ref[:, 0:1] + dg_ref[:, 1:2] + 1.0
    dinv = lax.rsqrt(jnp.maximum(dsum, 1.0))
    dv_ref[...] = dinv
    sc = dinv * (1.0 / BAG)
    e = e_ref[...]
    h_ref[0] = jnp.dot(e, w1_ref[...], preferred_element_type=jnp.float32) * sc
    h_ref[1] = jnp.dot(e, w2_ref[...], preferred_element_type=jnp.float32) * sc


def _tc1(emb, degt, w11, w21):
    return pl.pallas_call(
        _tc1_body,
        grid=(GRID,),
        in_specs=[
            pl.BlockSpec((R, IN_CH), lambda i: (i, 0)),
            pl.BlockSpec((R, 2), lambda i: (i, 0)),
            pl.BlockSpec((IN_CH, HID), lambda i: (0, 0)),
            pl.BlockSpec((IN_CH, HID), lambda i: (0, 0)),
        ],
        out_specs=[
            pl.BlockSpec((NC, R, HID), lambda i: (0, i, 0)),
            pl.BlockSpec((R, 1), lambda i: (i, 0)),
        ],
        out_shape=[
            jax.ShapeDtypeStruct((NC, N, HID), jnp.float32),
            jax.ShapeDtypeStruct((N, 1), jnp.float32),
        ],
    )(emb, degt, w11, w21)


def _prelu(t, a):
    return jnp.where(t > 0, t, a * t)


def _tc2_body(acc_ref, h_ref, dv_ref, b_ref,
              b1_ref, a1_ref, w1_ref, b2_ref, a2_ref, w2_ref,
              hn_ref, p1_ref, p2_ref):
    i = pl.program_id(0)
    dinv = dv_ref[...]
    bb = b_ref[...].reshape(1, R)
    oh = (lax.broadcasted_iota(jnp.int32, (NUM_GRAPHS, R), 0)
          == bb).astype(jnp.float32)

    z1 = _prelu((acc_ref[0] + h_ref[0]) * dinv + b1_ref[...], a1_ref[...])
    hn_ref[0] = jnp.dot(z1, w1_ref[...], preferred_element_type=jnp.float32) * dinv

    z2 = _prelu((acc_ref[1] + h_ref[1]) * dinv + b2_ref[...], a2_ref[...])
    hn_ref[1] = jnp.dot(z2, w2_ref[...], preferred_element_type=jnp.float32) * dinv

    @pl.when(i == 0)
    def _():
        p1_ref[...] = jnp.zeros_like(p1_ref)
        p2_ref[...] = jnp.zeros_like(p2_ref)

    p1_ref[...] += jnp.dot(oh, z1, preferred_element_type=jnp.float32)
    p2_ref[...] += jnp.dot(oh, z2, preferred_element_type=jnp.float32)


def _tc2(acc, h, dinv, batch2d, b1, a1, w1, b2, a2, w2):
    full = lambda shape: pl.BlockSpec(shape, lambda i: tuple(0 for _ in shape))
    return pl.pallas_call(
        _tc2_body,
        grid=(GRID,),
        in_specs=[
            pl.BlockSpec((NC, R, HID), lambda i: (0, i, 0)),
            pl.BlockSpec((NC, R, HID), lambda i: (0, i, 0)),
            pl.BlockSpec((R, 1), lambda i: (i, 0)),
            pl.BlockSpec((1, 1, R), lambda i: (i, 0, 0)),
            full((1, HID)), full((1, HID)), full((HID, HID)),
            full((1, HID)), full((1, HID)), full((HID, HID)),
        ],
        out_specs=[
            pl.BlockSpec((NC, R, HID), lambda i: (0, i, 0)),
            full((NUM_GRAPHS, HID)),
            full((NUM_GRAPHS, HID)),
        ],
        out_shape=[
            jax.ShapeDtypeStruct((NC, N, HID), jnp.float32),
            jax.ShapeDtypeStruct((NUM_GRAPHS, HID), jnp.float32),
            jax.ShapeDtypeStruct((NUM_GRAPHS, HID), jnp.float32),
        ],
    )(acc, h, dinv, batch2d, b1, a1, w1, b2, a2, w2)


def _mlp_apply(z, w1, b1, aa1, w2, b2, aa2, w3, b3, aa3, ws, bs):
    h = _prelu(jnp.dot(z, w1, preferred_element_type=jnp.float32) + b1, aa1)
    h = _prelu(jnp.dot(h, w2, preferred_element_type=jnp.float32) + b2, aa2)
    h = _prelu(jnp.dot(h, w3, preferred_element_type=jnp.float32) + b3, aa3)
    return h + jnp.dot(z, ws, preferred_element_type=jnp.float32) + bs


def _tc3_body(acc_ref, h_ref, dv_ref, b_ref, b1_ref, a1_ref, b2_ref, a2_ref,
              p11_ref, p21_ref,
              mw1, mb1, ma1, mw2, mb2, ma2, mw3, mb3, ma3, mws, mbs,
              gw1, gb1, ga1, gw2, gb2, ga2, gw3, gb3, ga3, gws, gbs,
              lv1_ref, lv2_ref, gv1_ref, gv2_ref, p1_ref, p2_ref):
    i = pl.program_id(0)
    dinv = dv_ref[...]
    bb = b_ref[...].reshape(1, R)
    oh = (lax.broadcasted_iota(jnp.int32, (NUM_GRAPHS, R), 0)
          == bb).astype(jnp.float32)

    z1 = _prelu((acc_ref[0] + h_ref[0]) * dinv + b1_ref[...], a1_ref[...])
    z2 = _prelu((acc_ref[1] + h_ref[1]) * dinv + b2_ref[...], a2_ref[...])

    @pl.when(i == 0)
    def _():
        p1_ref[...] = jnp.zeros_like(p1_ref)
        p2_ref[...] = jnp.zeros_like(p2_ref)

    p1_ref[...] += jnp.dot(oh, z1, preferred_element_type=jnp.float32)
    p2_ref[...] += jnp.dot(oh, z2, preferred_element_type=jnp.float32)

    args = (mw1[...], mb1[...], ma1[...], mw2[...], mb2[...], ma2[...],
            mw3[...], mb3[...], ma3[...], mws[...], mbs[...])
    lv1_ref[...] = _mlp_apply(z1, *args)
    lv2_ref[...] = _mlp_apply(z2, *args)

    @pl.when(i == GRID - 1)
    def _():
        gargs = (gw1[...], gb1[...], ga1[...], gw2[...], gb2[...], ga2[...],
                 gw3[...], gb3[...], ga3[...], gws[...], gbs[...])
        g1 = jnp.concatenate([p11_ref[...], p1_ref[...]], axis=1)
        g2 = jnp.concatenate([p21_ref[...], p2_ref[...]], axis=1)
        gv1_ref[...] = _mlp_apply(g1, *gargs)
        gv2_ref[...] = _mlp_apply(g2, *gargs)


def _tc3(acc, h, dinv, batch2d, b1, a1, b2, a2, p11, p21, m, gm):
    full = lambda shape: pl.BlockSpec(shape, lambda i: tuple(0 for _ in shape))
    mlp_specs = lambda k: (
        full((k * HID, HID)), full((1, HID)), full((1, 1)),
        full((HID, HID)), full((1, HID)), full((1, 1)),
        full((HID, HID)), full((1, HID)), full((1, 1)),
        full((k * HID, HID)), full((1, HID)))
    return pl.pallas_call(
        _tc3_body,
        grid=(GRID,),
        in_specs=[
            pl.BlockSpec((NC, R, HID), lambda i: (0, i, 0)),
            pl.BlockSpec((NC, R, HID), lambda i: (0, i, 0)),
            pl.BlockSpec((R, 1), lambda i: (i, 0)),
            pl.BlockSpec((1, 1, R), lambda i: (i, 0, 0)),
            full((1, HID)), full((1, HID)), full((1, HID)), full((1, HID)),
            full((NUM_GRAPHS, HID)), full((NUM_GRAPHS, HID)),
            *mlp_specs(1), *mlp_specs(2),
        ],
        out_specs=[
            pl.BlockSpec((R, HID), lambda i: (i, 0)),
            pl.BlockSpec((R, HID), lambda i: (i, 0)),
            full((NUM_GRAPHS, HID)),
            full((NUM_GRAPHS, HID)),
            full((NUM_GRAPHS, HID)),
            full((NUM_GRAPHS, HID)),
        ],
        out_shape=[
            jax.ShapeDtypeStruct((N, HID), jnp.float32),
            jax.ShapeDtypeStruct((N, HID), jnp.float32),
            jax.ShapeDtypeStruct((NUM_GRAPHS, HID), jnp.float32),
            jax.ShapeDtypeStruct((NUM_GRAPHS, HID), jnp.float32),
            jax.ShapeDtypeStruct((NUM_GRAPHS, HID), jnp.float32),
            jax.ShapeDtypeStruct((NUM_GRAPHS, HID), jnp.float32),
        ],
    )(acc, h, dinv, batch2d, b1, a1, b2, a2, p11, p21, *m, *gm)


def _mlp_args(p, in_ft):
    r = lambda v: v.reshape(1, -1)
    sc = lambda v: v.reshape(1, 1)
    return (p['W1'], r(p['b1']), sc(p['a1']),
            p['W2'], r(p['b2']), sc(p['a2']),
            p['W3'], r(p['b3']), sc(p['a3']),
            p['Ws'], r(p['bs']))


def kernel(x, edge_index, batch, params):
    p = params
    cb = p['codebook']

    xflat = x.reshape(N * BAG)
    src4d = edge_index[0].reshape(NS, NB, KP, PCH)
    dstflat = edge_index[1]
    dst4d = dstflat.reshape(NS, NB, KP, PCH)
    dst3d = dstflat.reshape(NC * NS * NBE, KB, CH)
    batchr = batch.reshape(GRID, 1, R)

    emb, degp = _sc_embdeg(xflat, dst3d, cb)
    degt = degp.reshape(NC, N).T

    g1, g2 = p['gnn1'], p['gnn2']
    (w11, b11), (w12, b12) = g1['layers']
    (w21, b21), (w22, b22) = g2['layers']
    a1 = g1['a'].reshape(1, HID)
    a2 = g2['a'].reshape(1, HID)
    rb = lambda v: v.reshape(1, HID)

    h, dinv = _tc1(emb, degt, w11, w21)
    acc = _sc_prop(src4d, dst4d, h)
    hn, p11, p21 = _tc2(acc, h, dinv, batchr,
                        rb(b11), a1, w12, rb(b21), a2, w22)
    acc2 = _sc_prop(src4d, dst4d, hn)
    lv1, lv2, gv1, gv2, _, _ = _tc3(acc2, hn, dinv, batchr,
                                    rb(b12), a1, rb(b22), a2, p11, p21,
                                    _mlp_args(p['mlp1'], HID),
                                    _mlp_args(p['mlp2'], 2 * HID))
    return (lv1, gv1, lv2, gv2)


# speedup vs baseline: 1.0408x; 1.0275x over previous
import functools

import jax
import jax.numpy as jnp
import numpy as np
from jax import lax
from jax.experimental import pallas as pl
from jax.experimental.pallas import tpu as pltpu
from jax.experimental.pallas import tpu_sc as plsc

N = 10000
E = 320000
BAG = 32
IN_CH = 128
HID = 128
NUM_GRAPHS = 16

NC = 2
NS = 16
CH = 80

EMB_CHUNKS = (N * BAG) // (NC * NS * CH)

PCH = 125
PROP_CHUNKS = E // (NS * PCH)

KB = 25
KP = 20
NB = PROP_CHUNKS // KP
NBE = EMB_CHUNKS // KB

R = 1000
GRID = N // R

_MESH = plsc.VectorSubcoreMesh(core_axis_name="c", subcore_axis_name="s")


def _zero_zbuf(zbuf):
    nrow = zbuf.shape[0]

    def body(i, _):
        for k in range(zbuf.shape[1] // 16):
            zbuf[i, pl.ds(k * 16, 16)] = jnp.zeros((16,), jnp.float32)
        return 0

    lax.fori_loop(0, nrow, body, 0)


@functools.partial(
    pl.kernel,
    mesh=_MESH,
    out_type=[
        jax.ShapeDtypeStruct((N, IN_CH), jnp.float32),
        jax.ShapeDtypeStruct((NC * N,), jnp.float32),
    ],
    scratch_types=[
        pltpu.VMEM((KB * CH,), jnp.int32),
        pltpu.VMEM((KB, CH), jnp.int32),
        pltpu.VMEM((CH,), jnp.int32),
        pltpu.VMEM((CH,), jnp.float32),
        pltpu.VMEM((CH, IN_CH), jnp.float32),
        pltpu.VMEM((CH, IN_CH), jnp.float32),
        pltpu.VMEM((2000,), jnp.float32),
        pltpu.VMEM_SHARED((N // NC, IN_CH), jnp.float32),
        pltpu.VMEM_SHARED((N,), jnp.float32),
        pltpu.SemaphoreType.DMA,
        pltpu.SemaphoreType.DMA,
    ],
)
def _sc_embdeg(xflat_hbm, dst3d_hbm, cb_hbm, emb_hbm, degp_hbm,
               bagidx_v, dstidx_v, sid_v, ones_v, rowsa_v, rowsb_v, zbuf1_v,
               emb_sh, deg_sh, sema, semb):
    c = lax.axis_index("c")
    s = lax.axis_index("s")
    rows_per_tile = N * BAG // (NC * NS)

    _zero_zbuf(rowsa_v)

    def z1(i, _):
        zbuf1_v[pl.ds(i * 16, 16)] = jnp.zeros((16,), jnp.float32)
        return 0

    lax.fori_loop(0, 125, z1, 0)

    def o1(i, _):
        ones_v[pl.ds(i * 16, 16)] = jnp.full((16,), 1.0, jnp.float32)
        return 0

    lax.fori_loop(0, CH // 16, o1, 0)

    zsrc = rowsa_v.at[pl.ds(0, 40)]
    for b in range(8):
        blk = s * 8 + b

        @pl.when(blk < 125)
        def _():
            pltpu.sync_copy(zsrc, emb_sh.at[pl.ds(blk * 40, 40)])

    @pl.when(s < 5)
    def _():
        pltpu.sync_copy(zbuf1_v, deg_sh.at[pl.ds(s * 2000, 2000)])

    plsc.subcore_barrier()

    row_off = s * rows_per_tile
    flat_off = (c * NS + s) * rows_per_tile

    def gather(j, rows, sem):
        pltpu.async_copy(cb_hbm.at[bagidx_v.at[pl.ds(j * CH, CH)]], rows, sem)

    def drain_scatter(b, j, rows, sem):
        pltpu.make_async_copy(cb_hbm.at[bagidx_v.at[pl.ds(j * CH, CH)]], rows,
                              sem).wait()
        base = row_off + (b * KB + j) * CH
        for k in range(CH // 16):
            ids = (base + k * 16 + lax.iota(jnp.int32, 16)) >> 5
            sid_v[pl.ds(k * 16, 16)] = ids
        pltpu.sync_copy(rows, emb_sh.at[sid_v], add=True)

    def eblock(b, _):
        pltpu.sync_copy(
            xflat_hbm.at[pl.ds(flat_off + b * (KB * CH), KB * CH)], bagidx_v)

        gather(0, rowsa_v, sema)

        def body(t, _):
            gather(2 * t + 1, rowsb_v, semb)
            drain_scatter(b, 2 * t, rowsa_v, sema)
            gather(2 * t + 2, rowsa_v, sema)
            drain_scatter(b, 2 * t + 1, rowsb_v, semb)
            return 0

        lax.fori_loop(0, KB // 2, body, 0)
        drain_scatter(b, KB - 1, rowsa_v, sema)
        return 0

    lax.fori_loop(0, NBE, eblock, 0)

    def dblock(b, _):
        pltpu.sync_copy(dst3d_hbm.at[(c * NS + s) * NBE + b], dstidx_v)

        def dissue(j, _):
            pltpu.async_copy(ones_v, deg_sh.at[dstidx_v.at[j]], sema, add=True)
            return 0

        lax.fori_loop(0, KB, dissue, 0)

        def ddrain(j, _):
            pltpu.make_async_copy(ones_v, deg_sh.at[dstidx_v.at[j]],
                                  sema).wait()
            return 0

        lax.fori_loop(0, KB, ddrain, 0)
        return 0

    lax.fori_loop(0, NBE, dblock, 0)

    plsc.subcore_barrier()

    for b in range(2):
        blk = s * 2 + b

        @pl.when(blk < 25)
        def _():
            pltpu.sync_copy(emb_sh.at[pl.ds(blk * 200, 200)],
                            emb_hbm.at[pl.ds(c * (N // NC) + blk * 200, 200)])

    @pl.when(s < 5)
    def _():
        pltpu.sync_copy(deg_sh.at[pl.ds(s * 2000, 2000)], zbuf1_v)
        pltpu.sync_copy(zbuf1_v, degp_hbm.at[pl.ds(c * N + s * 2000, 2000)])


@functools.partial(
    pl.kernel,
    mesh=_MESH,
    out_type=jax.ShapeDtypeStruct((NC, N, HID), jnp.float32),
    scratch_types=[
        pltpu.VMEM((2, KP, PCH), jnp.int32),
        pltpu.VMEM((2, KP, PCH), jnp.int32),
        pltpu.VMEM((PCH, HID), jnp.float32),
        pltpu.VMEM((PCH, HID), jnp.float32),
        pltpu.VMEM_SHARED((N, HID), jnp.float32),
        pltpu.SemaphoreType.DMA,
        pltpu.SemaphoreType.DMA,
        pltpu.SemaphoreType.DMA,
    ],
)
def _sc_prop(src4d_hbm, dst4d_hbm, h_hbm, acc_hbm,
             sidx_v, didx_v, rowsa_v, rowsb_v, acc_sh, sema, semb, semi):
    c = lax.axis_index("c")
    s = lax.axis_index("s")

    _zero_zbuf(rowsa_v)
    zsrc = rowsa_v.at[pl.ds(0, 80)]
    for b in range(8):
        blk = s * 8 + b

        @pl.when(blk < 125)
        def _():
            pltpu.sync_copy(zsrc, acc_sh.at[pl.ds(blk * 80, 80)])

    plsc.subcore_barrier()

    hc = h_hbm.at[c]

    def gather(p, j, rows, sem):
        pltpu.async_copy(hc.at[sidx_v.at[p].at[j]], rows, sem)

    def drain_scatter(p, j, rows, sem):
        pltpu.make_async_copy(hc.at[sidx_v.at[p].at[j]], rows, sem).wait()
        pltpu.sync_copy(rows, acc_sh.at[didx_v.at[p].at[j]], add=True)

    def load_idx(b, p):
        pltpu.async_copy(src4d_hbm.at[s].at[b], sidx_v.at[p], semi)
        pltpu.async_copy(dst4d_hbm.at[s].at[b], didx_v.at[p], semi)

    def wait_idx(b, p):
        pltpu.make_async_copy(src4d_hbm.at[s].at[b], sidx_v.at[p], semi).wait()
        pltpu.make_async_copy(dst4d_hbm.at[s].at[b], didx_v.at[p], semi).wait()

    load_idx(0, 0)

    def block(b, _):
        p = b % 2
        wait_idx(b, p)

        gather(p, 0, rowsa_v, sema)

        @pl.when(b + 1 < NB)
        def _():
            load_idx(b + 1, 1 - p)

        def body(t, _):
            gather(p, 2 * t + 1, rowsb_v, semb)
            drain_scatter(p, 2 * t, rowsa_v, sema)
            gather(p, 2 * t + 2, rowsa_v, sema)
            drain_scatter(p, 2 * t + 1, rowsb_v, semb)
            return 0

        lax.fori_loop(0, KP // 2 - 1, body, 0)
        gather(p, KP - 1, rowsb_v, semb)
        drain_scatter(p, KP - 2, rowsa_v, sema)
        drain_scatter(p, KP - 1, rowsb_v, semb)
        return 0

    lax.fori_loop(0, NB, block, 0)

    plsc.subcore_barrier()

    for b in range(4):
        blk = s * 4 + b

        @pl.when(blk < 50)
        def _():
            pltpu.sync_copy(acc_sh.at[pl.ds(blk * 200, 200)],
                            acc_hbm.at[c, pl.ds(blk * 200, 200)])


def _tc1_body(e_ref, dg_ref, w1_ref, w2_ref, h_ref, dv_ref):
    dsum = dg_ref[:, 0:1] + dg_ref[:, 1:2] + 1.0
    dinv = lax.rsqrt(jnp.maximum(dsum, 1.0))
    dv_ref[...] = dinv
    sc = dinv * (1.0 / BAG)
    e = e_ref[...]
    h_ref[0] = jnp.dot(e, w1_ref[...], preferred_element_type=jnp.float32) * sc
    h_ref[1] = jnp.dot(e, w2_ref[...], preferred_element_type=jnp.float32) * sc


def _tc1(emb, degt, w11, w21):
    return pl.pallas_call(
        _tc1_body,
        grid=(GRID,),
        in_specs=[
            pl.BlockSpec((R, IN_CH), lambda i: (i, 0)),
            pl.BlockSpec((R, 2), lambda i: (i, 0)),
            pl.BlockSpec((IN_CH, HID), lambda i: (0, 0)),
            pl.BlockSpec((IN_CH, HID), lambda i: (0, 0)),
        ],
        out_specs=[
            pl.BlockSpec((NC, R, HID), lambda i: (0, i, 0)),
            pl.BlockSpec((R, 1), lambda i: (i, 0)),
        ],
        out_shape=[
            jax.ShapeDtypeStruct((NC, N, HID), jnp.float32),
            jax.ShapeDtypeStruct((N, 1), jnp.float32),
        ],
    )(emb, degt, w11, w21)


def _prelu(t, a):
    return jnp.where(t > 0, t, a * t)


def _tc2_body(acc_ref, h_ref, dv_ref, b_ref,
              b1_ref, a1_ref, w1_ref, b2_ref, a2_ref, w2_ref,
              hn_ref, p1_ref, p2_ref):
    i = pl.program_id(0)
    dinv = dv_ref[...]
    bb = b_ref[...].reshape(1, R)
    oh = (lax.broadcasted_iota(jnp.int32, (NUM_GRAPHS, R), 0)
          == bb).astype(jnp.float32)

    z1 = _prelu((acc_ref[0] + h_ref[0]) * dinv + b1_ref[...], a1_ref[...])
    hn_ref[0] = jnp.dot(z1, w1_ref[...], preferred_element_type=jnp.float32) * dinv

    z2 = _prelu((acc_ref[1] + h_ref[1]) * dinv + b2_ref[...], a2_ref[...])
    hn_ref[1] = jnp.dot(z2, w2_ref[...], preferred_element_type=jnp.float32) * dinv

    @pl.when(i == 0)
    def _():
        p1_ref[...] = jnp.zeros_like(p1_ref)
        p2_ref[...] = jnp.zeros_like(p2_ref)

    p1_ref[...] += jnp.dot(oh, z1, preferred_element_type=jnp.float32)
    p2_ref[...] += jnp.dot(oh, z2, preferred_element_type=jnp.float32)


def _tc2(acc, h, dinv, batch2d, b1, a1, w1, b2, a2, w2):
    full = lambda shape: pl.BlockSpec(shape, lambda i: tuple(0 for _ in shape))
    return pl.pallas_call(
        _tc2_body,
        grid=(GRID,),
        in_specs=[
            pl.BlockSpec((NC, R, HID), lambda i: (0, i, 0)),
            pl.BlockSpec((NC, R, HID), lambda i: (0, i, 0)),
            pl.BlockSpec((R, 1), lambda i: (i, 0)),
            pl.BlockSpec((1, 1, R), lambda i: (i, 0, 0)),
            full((1, HID)), full((1, HID)), full((HID, HID)),
            full((1, HID)), full((1, HID)), full((HID, HID)),
        ],
        out_specs=[
            pl.BlockSpec((NC, R, HID), lambda i: (0, i, 0)),
            full((NUM_GRAPHS, HID)),
            full((NUM_GRAPHS, HID)),
        ],
        out_shape=[
            jax.ShapeDtypeStruct((NC, N, HID), jnp.float32),
            jax.ShapeDtypeStruct((NUM_GRAPHS, HID), jnp.float32),
            jax.ShapeDtypeStruct((NUM_GRAPHS, HID), jnp.float32),
        ],
    )(acc, h, dinv, batch2d, b1, a1, w1, b2, a2, w2)


def _mlp_apply(z, w1, b1, aa1, w2, b2, aa2, w3, b3, aa3, ws, bs):
    h = _prelu(jnp.dot(z, w1, preferred_element_type=jnp.float32) + b1, aa1)
    h = _prelu(jnp.dot(h, w2, preferred_element_type=jnp.float32) + b2, aa2)
    h = _prelu(jnp.dot(h, w3, preferred_element_type=jnp.float32) + b3, aa3)
    return h + jnp.dot(z, ws, preferred_element_type=jnp.float32) + bs


def _tc3_body(acc_ref, h_ref, dv_ref, b_ref, b1_ref, a1_ref, b2_ref, a2_ref,
              p11_ref, p21_ref,
              mw1, mb1, ma1, mw2, mb2, ma2, mw3, mb3, ma3, mws, mbs,
              gw1, gb1, ga1, gw2, gb2, ga2, gw3, gb3, ga3, gws, gbs,
              lv1_ref, lv2_ref, gv1_ref, gv2_ref, p1_ref, p2_ref):
    i = pl.program_id(0)
    dinv = dv_ref[...]
    bb = b_ref[...].reshape(1, R)
    oh = (lax.broadcasted_iota(jnp.int32, (NUM_GRAPHS, R), 0)
          == bb).astype(jnp.float32)

    z1 = _prelu((acc_ref[0] + h_ref[0]) * dinv + b1_ref[...], a1_ref[...])
    z2 = _prelu((acc_ref[1] + h_ref[1]) * dinv + b2_ref[...], a2_ref[...])

    @pl.when(i == 0)
    def _():
        p1_ref[...] = jnp.zeros_like(p1_ref)
        p2_ref[...] = jnp.zeros_like(p2_ref)

    p1_ref[...] += jnp.dot(oh, z1, preferred_element_type=jnp.float32)
    p2_ref[...] += jnp.dot(oh, z2, preferred_element_type=jnp.float32)

    args = (mw1[...], mb1[...], ma1[...], mw2[...], mb2[...], ma2[...],
            mw3[...], mb3[...], ma3[...], mws[...], mbs[...])
    lv1_ref[...] = _mlp_apply(z1, *args)
    lv2_ref[...] = _mlp_apply(z2, *args)

    @pl.when(i == GRID - 1)
    def _():
        gargs = (gw1[...], gb1[...], ga1[...], gw2[...], gb2[...], ga2[...],
                 gw3[...], gb3[...], ga3[...], gws[...], gbs[...])
        g1 = jnp.concatenate([p11_ref[...], p1_ref[...]], axis=1)
        g2 = jnp.concatenate([p21_ref[...], p2_ref[...]], axis=1)
        gv1_ref[...] = _mlp_apply(g1, *gargs)
        gv2_ref[...] = _mlp_apply(g2, *gargs)


def _tc3(acc, h, dinv, batch2d, b1, a1, b2, a2, p11, p21, m, gm):
    full = lambda shape: pl.BlockSpec(shape, lambda i: tuple(0 for _ in shape))
    mlp_specs = lambda k: (
        full((k * HID, HID)), full((1, HID)), full((1, 1)),
        full((HID, HID)), full((1, HID)), full((1, 1)),
        full((HID, HID)), full((1, HID)), full((1, 1)),
        full((k * HID, HID)), full((1, HID)))
    return pl.pallas_call(
        _tc3_body,
        grid=(GRID,),
        in_specs=[
            pl.BlockSpec((NC, R, HID), lambda i: (0, i, 0)),
            pl.BlockSpec((NC, R, HID), lambda i: (0, i, 0)),
            pl.BlockSpec((R, 1), lambda i: (i, 0)),
            pl.BlockSpec((1, 1, R), lambda i: (i, 0, 0)),
            full((1, HID)), full((1, HID)), full((1, HID)), full((1, HID)),
            full((NUM_GRAPHS, HID)), full((NUM_GRAPHS, HID)),
            *mlp_specs(1), *mlp_specs(2),
        ],
        out_specs=[
            pl.BlockSpec((R, HID), lambda i: (i, 0)),
            pl.BlockSpec((R, HID), lambda i: (i, 0)),
            full((NUM_GRAPHS, HID)),
            full((NUM_GRAPHS, HID)),
            full((NUM_GRAPHS, HID)),
            full((NUM_GRAPHS, HID)),
        ],
        out_shape=[
            jax.ShapeDtypeStruct((N, HID), jnp.float32),
            jax.ShapeDtypeStruct((N, HID), jnp.float32),
            jax.ShapeDtypeStruct((NUM_GRAPHS, HID), jnp.float32),
            jax.ShapeDtypeStruct((NUM_GRAPHS, HID), jnp.float32),
            jax.ShapeDtypeStruct((NUM_GRAPHS, HID), jnp.float32),
            jax.ShapeDtypeStruct((NUM_GRAPHS, HID), jnp.float32),
        ],
    )(acc, h, dinv, batch2d, b1, a1, b2, a2, p11, p21, *m, *gm)


def _mlp_args(p, in_ft):
    r = lambda v: v.reshape(1, -1)
    sc = lambda v: v.reshape(1, 1)
    return (p['W1'], r(p['b1']), sc(p['a1']),
            p['W2'], r(p['b2']), sc(p['a2']),
            p['W3'], r(p['b3']), sc(p['a3']),
            p['Ws'], r(p['bs']))


def kernel(x, edge_index, batch, params):
    p = params
    cb = p['codebook']

    xflat = x.reshape(N * BAG)
    src4d = edge_index[0].reshape(NS, NB, KP, PCH)
    dstflat = edge_index[1]
    dst4d = dstflat.reshape(NS, NB, KP, PCH)
    dst3d = dstflat.reshape(NC * NS * NBE, KB, CH)
    batchr = batch.reshape(GRID, 1, R)

    emb, degp = _sc_embdeg(xflat, dst3d, cb)
    degt = degp.reshape(NC, N).T

    g1, g2 = p['gnn1'], p['gnn2']
    (w11, b11), (w12, b12) = g1['layers']
    (w21, b21), (w22, b22) = g2['layers']
    a1 = g1['a'].reshape(1, HID)
    a2 = g2['a'].reshape(1, HID)
    rb = lambda v: v.reshape(1, HID)

    h, dinv = _tc1(emb, degt, w11, w21)
    acc = _sc_prop(src4d, dst4d, h)
    hn, p11, p21 = _tc2(acc, h, dinv, batchr,
                        rb(b11), a1, w12, rb(b21), a2, w22)
    acc2 = _sc_prop(src4d, dst4d, hn)
    lv1, lv2, gv1, gv2, _, _ = _tc3(acc2, hn, dinv, batchr,
                                    rb(b12), a1, rb(b22), a2, p11, p21,
                                    _mlp_args(p['mlp1'], HID),
                                    _mlp_args(p['mlp2'], 2 * HID))
    return (lv1, gv1, lv2, gv2)
